# baseline (speedup 1.0000x reference)
import jax
import jax.numpy as jnp
from jax import lax
from jax.experimental import pallas as pl
from jax.experimental.pallas import tpu as pltpu
from jax.experimental.pallas import tpu_sc as plsc

_NUM_CLASS = 13164
_VOCAB = 49408
_D = 512
_SEQ = 77
_B = 1024
_PRE = 12
_NCTX = 4
_SUF = 20
_SUF_START = 57
_ROWS = _PRE + _NCTX + _SUF
_ZROWS = _SEQ - _ROWS

_NC = 2
_NS = 16
_NW = _NC * _NS
_BPW = _B // _NW

_IW = 48
_PRE_PAD = 16
_SUF_PAD = 24
_CLS_PAD = 8


def _sc_body(idx_hbm, table_hbm, cls_hbm, out_hbm, idx_v, buf_v, zbuf_v, sem):
    wid = lax.axis_index("s") * _NC + lax.axis_index("c")
    base = wid * _BPW

    pltpu.sync_copy(idx_hbm.at[pl.ds(base, _BPW)], idx_v)

    def _zero(k, _):
        r = k // (_D // 16)
        c = k % (_D // 16)
        zbuf_v[r, pl.ds(c * 16, 16)] = jnp.zeros((16,), jnp.float32)
        return 0

    lax.fori_loop(0, _ZROWS * (_D // 16), _zero, 0)

    def _per_row(i, _):
        b = base + i
        g1 = pltpu.async_copy(
            table_hbm.at[idx_v.at[i, pl.ds(0, _PRE_PAD)]],
            buf_v.at[pl.ds(0, _PRE_PAD)], sem)
        g2 = pltpu.async_copy(
            table_hbm.at[idx_v.at[i, pl.ds(16, _SUF_PAD)]],
            buf_v.at[pl.ds(16, _SUF_PAD)], sem)
        g3 = pltpu.async_copy(
            cls_hbm.at[idx_v.at[i, pl.ds(40, _CLS_PAD)]],
            buf_v.at[pl.ds(40, _CLS_PAD)], sem)
        g1.wait()
        g2.wait()
        g3.wait()
        pltpu.sync_copy(buf_v.at[pl.ds(0, _PRE)],
                        out_hbm.at[b, pl.ds(0, _PRE)])
        pltpu.sync_copy(buf_v.at[pl.ds(40, _NCTX)],
                        out_hbm.at[b, pl.ds(_PRE, _NCTX)])
        pltpu.sync_copy(buf_v.at[pl.ds(16, _SUF)],
                        out_hbm.at[b, pl.ds(_PRE + _NCTX, _SUF)])
        pltpu.sync_copy(zbuf_v, out_hbm.at[b, pl.ds(_ROWS, _ZROWS)])
        return 0

    lax.fori_loop(0, _BPW, _per_row, 0)


def kernel(vehicle_ids, tokenized_prompts, token_table, cls_ctx):
    tp = tokenized_prompts.astype(jnp.int32)
    vid = vehicle_ids.astype(jnp.int32)

    idx = jnp.zeros((_B, _IW), jnp.int32)
    idx = idx.at[:, 0:_PRE].set(tp[:, 0:_PRE])
    idx = idx.at[:, 16:16 + _SUF].set(tp[:, _SUF_START:_SEQ])
    idx = idx.at[:, 40:40 + _NCTX].set(
        vid[:, None] * _NCTX + jnp.arange(_NCTX, dtype=jnp.int32))

    cls_flat = cls_ctx.reshape(_NUM_CLASS * _NCTX, _D)

    mesh = plsc.VectorSubcoreMesh(
        core_axis_name="c", subcore_axis_name="s",
        num_cores=_NC, num_subcores=_NS)

    run = pl.kernel(
        _sc_body,
        out_type=jax.ShapeDtypeStruct((_B, _SEQ, _D), jnp.float32),
        mesh=mesh,
        scratch_types=[
            pltpu.VMEM((_BPW, _IW), jnp.int32),
            pltpu.VMEM((_PRE_PAD + _SUF_PAD + _CLS_PAD, _D), jnp.float32),
            pltpu.VMEM((_ZROWS, _D), jnp.float32),
            pltpu.SemaphoreType.DMA,
        ],
        compiler_params=pltpu.CompilerParams(use_tc_tiling_on_sc=False),
    )
    return run(idx, token_table, cls_flat)


# speedup vs baseline: 1.8975x; 1.8975x over previous
import jax
import jax.numpy as jnp
from jax import lax
from jax.experimental import pallas as pl
from jax.experimental.pallas import tpu as pltpu
from jax.experimental.pallas import tpu_sc as plsc

_NUM_CLASS = 13164
_D = 512
_SEQ = 77
_B = 1024
_PRE = 12
_NCTX = 4
_SUF = 20
_SUF_START = 57
_ROWS = _PRE + _NCTX + _SUF
_ZROWS = _SEQ - _ROWS

_NC = 2
_NS = 16
_NW = _NC * _NS
_BPW = _B // _NW

_IW = 40
_TOK = _PRE + _SUF
_CLS_PAD = 8

_NSLOT = 4
_ROUNDS = _BPW // _NSLOT


def _issue_gathers(idx_v, table_hbm, cls_hbm, buf, gsem, i):
    gt = pltpu.async_copy(
        table_hbm.at[idx_v.at[i, pl.ds(0, _TOK)]],
        buf.at[pl.ds(0, _TOK)], gsem)
    gc = pltpu.async_copy(
        cls_hbm.at[idx_v.at[i, pl.ds(_TOK, _CLS_PAD)]],
        buf.at[pl.ds(_TOK, _CLS_PAD)], gsem)
    return gt, gc


def _issue_out(buf, zbuf, out_hbm, osem, b):
    o1 = pltpu.async_copy(buf.at[pl.ds(0, _PRE)],
                          out_hbm.at[b, pl.ds(0, _PRE)], osem)
    o2 = pltpu.async_copy(buf.at[pl.ds(_TOK, _NCTX)],
                          out_hbm.at[b, pl.ds(_PRE, _NCTX)], osem)
    o3 = pltpu.async_copy(buf.at[pl.ds(_PRE, _SUF)],
                          out_hbm.at[b, pl.ds(_PRE + _NCTX, _SUF)], osem)
    o4 = pltpu.async_copy(zbuf, out_hbm.at[b, pl.ds(_ROWS, _ZROWS)], osem)
    return o1, o2, o3, o4


def _sc_body(idx_hbm, table_hbm, cls_hbm, out_hbm,
             idx_v, b0, b1, b2, b3, zbuf, isem,
             g0, g1, g2, g3, s0, s1, s2, s3):
    bufs = (b0, b1, b2, b3)
    gsems = (g0, g1, g2, g3)
    osems = (s0, s1, s2, s3)

    wid = lax.axis_index("s") * _NC + lax.axis_index("c")
    base = wid * _BPW

    pltpu.sync_copy(idx_hbm.at[pl.ds(base, _BPW)], idx_v)

    def _zero(k, _):
        r = k // (_D // 16)
        c = k % (_D // 16)
        zbuf[r, pl.ds(c * 16, 16)] = jnp.zeros((16,), jnp.float32)
        return 0

    lax.fori_loop(0, _ZROWS * (_D // 16), _zero, 0)

    for s in range(_NSLOT):
        gt, gc = _issue_gathers(idx_v, table_hbm, cls_hbm, bufs[s],
                                gsems[s], s)

    def _round(k, _):
        descs = []
        for s in range(_NSLOT):
            i = k * _NSLOT + s
            b = base + i
            pltpu.make_async_copy(
                table_hbm.at[idx_v.at[i, pl.ds(0, _TOK)]],
                bufs[s].at[pl.ds(0, _TOK)], gsems[s]).wait()
            pltpu.make_async_copy(
                cls_hbm.at[idx_v.at[i, pl.ds(_TOK, _CLS_PAD)]],
                bufs[s].at[pl.ds(_TOK, _CLS_PAD)], gsems[s]).wait()
            descs.append(_issue_out(bufs[s], zbuf, out_hbm, osems[s], b))
        for s in range(_NSLOT):
            i_next = (k + 1) * _NSLOT + s
            for d in descs[s]:
                d.wait()
            _issue_gathers(idx_v, table_hbm, cls_hbm, bufs[s],
                           gsems[s], i_next)
        return 0

    lax.fori_loop(0, _ROUNDS - 1, _round, 0)

    for s in range(_NSLOT):
        i = (_ROUNDS - 1) * _NSLOT + s
        b = base + i
        pltpu.make_async_copy(
            table_hbm.at[idx_v.at[i, pl.ds(0, _TOK)]],
            bufs[s].at[pl.ds(0, _TOK)], gsems[s]).wait()
        pltpu.make_async_copy(
            cls_hbm.at[idx_v.at[i, pl.ds(_TOK, _CLS_PAD)]],
            bufs[s].at[pl.ds(_TOK, _CLS_PAD)], gsems[s]).wait()
        for d in _issue_out(bufs[s], zbuf, out_hbm, osems[s], b):
            d.wait()


def kernel(vehicle_ids, tokenized_prompts, token_table, cls_ctx):
    tp = tokenized_prompts.astype(jnp.int32)
    vid = vehicle_ids.astype(jnp.int32)

    cls_rows = vid[:, None] * _NCTX + jnp.arange(_NCTX, dtype=jnp.int32)
    idx = jnp.concatenate(
        [tp[:, 0:_PRE], tp[:, _SUF_START:_SEQ],
         cls_rows, cls_rows], axis=1)

    cls_flat = cls_ctx.reshape(_NUM_CLASS * _NCTX, _D)

    mesh = plsc.VectorSubcoreMesh(
        core_axis_name="c", subcore_axis_name="s",
        num_cores=_NC, num_subcores=_NS)

    slot = pltpu.VMEM((_TOK + _CLS_PAD, _D), jnp.float32)
    run = pl.kernel(
        _sc_body,
        out_type=jax.ShapeDtypeStruct((_B, _SEQ, _D), jnp.float32),
        mesh=mesh,
        scratch_types=[
            pltpu.VMEM((_BPW, _IW), jnp.int32),
            slot, slot, slot, slot,
            pltpu.VMEM((_ZROWS, _D), jnp.float32),
            pltpu.SemaphoreType.DMA,
            pltpu.SemaphoreType.DMA, pltpu.SemaphoreType.DMA,
            pltpu.SemaphoreType.DMA, pltpu.SemaphoreType.DMA,
            pltpu.SemaphoreType.DMA, pltpu.SemaphoreType.DMA,
            pltpu.SemaphoreType.DMA, pltpu.SemaphoreType.DMA,
        ],
        compiler_params=pltpu.CompilerParams(use_tc_tiling_on_sc=False),
    )
    return run(idx, token_table, cls_flat)


# speedup vs baseline: 5.5019x; 2.8995x over previous
import jax
import jax.numpy as jnp
from jax import lax
from jax.experimental import pallas as pl
from jax.experimental.pallas import tpu as pltpu
from jax.experimental.pallas import tpu_sc as plsc

_D = 512
_SEQ = 77
_B = 1024
_PRE = 12
_NCTX = 4
_SUF = 20
_SUF_START = 57
_ROWS = _PRE + _NCTX + _SUF
_TOK = _PRE + _SUF

_NC = 2
_NS = 16
_NW = _NC * _NS
_BPW = _B // _NW

_CCH = 8
_NSLOT = 4
_ASM_G = 8


def _sc_body(idxt_hbm, vid_hbm, table_hbm, cls_hbm, z_hbm,
             out_t, cls_out, idxt_v, vid_v, zbuf, b0, b1, b2, b3, cbuf,
             csem, zsem, g0, g1, g2, g3, s0, s1, s2, s3):
    bufs = (b0, b1, b2, b3)
    gsems = (g0, g1, g2, g3)
    osems = (s0, s1, s2, s3)

    wid = lax.axis_index("s") * _NC + lax.axis_index("c")
    base = wid * _BPW

    blk = (wid // 4) * 128
    col = (wid % 4) * _BPW
    pltpu.sync_copy(idxt_hbm.at[:, pl.ds(blk, 128)], idxt_v)
    pltpu.sync_copy(vid_hbm.at[pl.ds(base, _BPW)], vid_v)
    pltpu.sync_copy(z_hbm, zbuf)

    def _zfire(j, _):
        pltpu.async_copy(zbuf, out_t.at[_ROWS + j, pl.ds(base, _BPW)], zsem)
        return 0

    lax.fori_loop(0, _SEQ - _ROWS, _zfire, 0)

    def _cls_chunk(j, _):
        c0 = base + j * _CCH
        pltpu.async_copy(
            cls_hbm.at[vid_v.at[pl.ds(j * _CCH, _CCH)]], cbuf, csem).wait()
        pltpu.sync_copy(cbuf, cls_out.at[pl.ds(c0, _CCH)])
        return 0

    lax.fori_loop(0, _BPW // _CCH, _cls_chunk, 0)

    def _pos_out(p):
        return p + jnp.where(p >= _PRE, _NCTX, 0)

    for s in range(_NSLOT):
        pltpu.async_copy(
            table_hbm.at[idxt_v.at[s, pl.ds(col, _BPW)]], bufs[s], gsems[s])

    def _round(k, _):
        descs = []
        for s in range(_NSLOT):
            p = k * _NSLOT + s
            pltpu.make_async_copy(
                table_hbm.at[idxt_v.at[p, pl.ds(col, _BPW)]],
                bufs[s], gsems[s]).wait()
            descs.append(pltpu.async_copy(
                bufs[s], out_t.at[_pos_out(p), pl.ds(base, _BPW)], osems[s]))
        for s in range(_NSLOT):
            descs[s].wait()
            pltpu.async_copy(
                table_hbm.at[idxt_v.at[(k + 1) * _NSLOT + s, pl.ds(col, _BPW)]],
                bufs[s], gsems[s])
        return 0

    lax.fori_loop(0, _TOK // _NSLOT - 1, _round, 0)

    for s in range(_NSLOT):
        p = _TOK - _NSLOT + s
        pltpu.make_async_copy(
            table_hbm.at[idxt_v.at[p, pl.ds(col, _BPW)]],
            bufs[s], gsems[s]).wait()
        pltpu.async_copy(
            bufs[s], out_t.at[_pos_out(p), pl.ds(base, _BPW)], osems[s]).wait()

    def _zdrain(j, _):
        pltpu.make_async_copy(
            zbuf, out_t.at[_ROWS + j, pl.ds(base, _BPW)], zsem).wait()
        return 0

    lax.fori_loop(0, _SEQ - _ROWS, _zdrain, 0)


def _patch_body(ot_ref, cls_ref, o_ref):
    for q in range(_NCTX):
        o_ref[q] = cls_ref[:, q, :]


def kernel(vehicle_ids, tokenized_prompts, token_table, cls_ctx):
    tp = tokenized_prompts.astype(jnp.int32)
    vid = vehicle_ids.astype(jnp.int32)
    idx_t = jnp.concatenate(
        [tp[:, :_PRE], tp[:, _SUF_START:_SEQ]], axis=1).T
    zeros = jnp.zeros((_BPW, _D), jnp.float32)

    mesh = plsc.VectorSubcoreMesh(
        core_axis_name="c", subcore_axis_name="s",
        num_cores=_NC, num_subcores=_NS)

    slot = pltpu.VMEM((_BPW, _D), jnp.float32)
    gather = pl.kernel(
        _sc_body,
        out_type=(jax.ShapeDtypeStruct((_SEQ, _B, _D), jnp.float32),
                  jax.ShapeDtypeStruct((_B, _NCTX, _D), jnp.float32)),
        mesh=mesh,
        scratch_types=[
            pltpu.VMEM((_TOK, 128), jnp.int32),
            pltpu.VMEM((_BPW,), jnp.int32),
            pltpu.VMEM((_BPW, _D), jnp.float32),
            slot, slot, slot, slot,
            pltpu.VMEM((_CCH, _NCTX, _D), jnp.float32),
            pltpu.SemaphoreType.DMA, pltpu.SemaphoreType.DMA,
            pltpu.SemaphoreType.DMA, pltpu.SemaphoreType.DMA,
            pltpu.SemaphoreType.DMA, pltpu.SemaphoreType.DMA,
            pltpu.SemaphoreType.DMA, pltpu.SemaphoreType.DMA,
            pltpu.SemaphoreType.DMA, pltpu.SemaphoreType.DMA,
        ],
        compiler_params=pltpu.CompilerParams(use_tc_tiling_on_sc=True),
    )
    out_t, sc_cls = gather(idx_t, vid, token_table, cls_ctx, zeros)

    patched = pl.pallas_call(
        _patch_body,
        grid=(_B // _ASM_G,),
        in_specs=[
            pl.BlockSpec(memory_space=pltpu.MemorySpace.HBM),
            pl.BlockSpec((_ASM_G, _NCTX, _D), lambda i: (i, 0, 0)),
        ],
        out_specs=pl.BlockSpec((_NCTX, _ASM_G, _D), lambda i: (3, i, 0)),
        out_shape=jax.ShapeDtypeStruct((_SEQ, _B, _D), jnp.float32),
        input_output_aliases={0: 0},
    )(out_t, sc_cls)

    return jnp.transpose(patched, (1, 0, 2))
